# baseline (speedup 1.0000x reference)
import jax
import jax.numpy as jnp
from jax.experimental import pallas as pl
from jax.experimental.pallas import tpu as pltpu

_BEAM = 10
_FC = 256


def _mm(a, b):
    return jax.lax.dot_general(a, b, (((1,), (0,)), ((), ())),
                               preferred_element_type=jnp.float32)


def _tm(a, b):
    return jax.lax.dot_general(a, b, (((0,), (0,)), ((), ())),
                               preferred_element_type=jnp.float32)


def _mmt(a, b):
    return jax.lax.dot_general(a, b, (((1,), (1,)), ((), ())),
                               preferred_element_type=jnp.float32)


def _prove_kernel(nb_s_ref, rel_ref, a1_ref, a2_ref, frT_ref,
                  fa1T_ref, fa2T_ref, entT_ref, w1_ref, w2_ref, out_ref,
                  s0_scr, dmin_scr):
    B, E, F = frT_ref.shape
    N = entT_ref.shape[2]
    inf = jnp.float32(jnp.inf)

    ones_row = jnp.ones((1, E), jnp.float32)
    lane_f = jax.lax.broadcasted_iota(jnp.int32, (1, F), 1)

    def sq(x):
        return jnp.sum(x * x)

    def d2_row(x, fT, fsq_row):
        return jnp.maximum((sq(x) + fsq_row) - 2.0 * _mm(x, fT), 0.0)

    def phase1_one(b):
        nb_b = nb_s_ref[b]
        rel_b = rel_ref[b]
        a1_b = a1_ref[b]
        a2_b = a2_ref[b]
        hop1_b = _mm(rel_b, w1_ref[...])
        frT_b = frT_ref[b]
        fa1T_b = fa1T_ref[b]
        fa2T_b = fa2T_ref[b]
        entT_b = entT_ref[b]
        valid_row = lane_f < nb_b

        fr2_row = jnp.sum(frT_b * frT_b, axis=0, keepdims=True)
        fa1sq_row = jnp.sum(fa1T_b * fa1T_b, axis=0, keepdims=True)
        fa2sq_row = jnp.sum(fa2T_b * fa2T_b, axis=0, keepdims=True)

        xf2r = _mm(jnp.concatenate([rel_b, hop1_b], axis=0), frT_b)
        d2_rel = jnp.maximum((sq(rel_b) + fr2_row) - 2.0 * xf2r[0:1], 0.0)
        d2_hop1 = jnp.maximum((sq(hop1_b) + fr2_row) - 2.0 * xf2r[1:2], 0.0)
        d2_a1 = d2_row(a1_b, fa1T_b, fa1sq_row)
        d2_a2 = d2_row(a2_b, fa2T_b, fa2sq_row)

        s0_row = jnp.maximum(jnp.maximum(d2_rel, d2_a1), d2_a2)
        s0_row = jnp.where(valid_row, s0_row, inf)

        cap1_row = jnp.maximum(d2_hop1, d2_a1)
        cap1_row = jnp.where(valid_row, cap1_row, inf)
        cap1_col = _tm(cap1_row, jnp.ones((1, 1), jnp.float32))

        e2_row = jnp.sum(entT_b * entT_b, axis=0, keepdims=True)
        fa2sq_col = _tm(fa2sq_row, jnp.ones((1, 1), jnp.float32))

        def chunk_min(t):
            xf = _tm(fa2T_b[:, t * _FC:(t + 1) * _FC], entT_b)
            d2 = jnp.maximum(
                (e2_row + fa2sq_col[t * _FC:(t + 1) * _FC]) - 2.0 * xf, 0.0)
            m = jnp.maximum(d2, cap1_col[t * _FC:(t + 1) * _FC])
            return jnp.min(m, axis=0, keepdims=True)

        s0_scr[b] = s0_row
        dmin_scr[b] = chunk_min(0)
        for t in range(1, F // _FC):
            @pl.when(nb_b > t * _FC)
            def _(t=t):
                dmin_scr[b] = jnp.minimum(dmin_scr[b], chunk_min(t))

    def phase1_body(b, carry):
        phase1_one(b)
        return carry

    jax.lax.fori_loop(0, B, phase1_body, 0)
    s0_all = jnp.concatenate([s0_scr[b] for b in range(B)], axis=0)
    dmin_all = jnp.concatenate([dmin_scr[b] for b in range(B)], axis=0)
    score0 = jnp.exp(-0.5 * jnp.min(s0_all, axis=1, keepdims=True))

    vals = jnp.exp(-0.5 * dmin_all)

    lane_n = jax.lax.broadcasted_iota(jnp.int32, (B, N), 1)
    ohs = []
    z_cols = []
    v = vals
    for _ in range(_BEAM):
        mv = jnp.max(v, axis=1, keepdims=True)
        idx = jnp.min(jnp.where(v == mv, lane_n, N), axis=1,
                      keepdims=True)
        oh = lane_n == idx
        v = jnp.where(oh, -inf, v)
        z_cols.append(mv)
        ohs.append(oh.astype(jnp.float32))

    m2_rows = []
    z_parts = []
    for b in range(B):
        nb_b = nb_s_ref[b]
        frT_b = frT_ref[b]
        fa1T_b = fa1T_ref[b]
        fa2T_b = fa2T_ref[b]
        entT_b = entT_ref[b]
        hop2_b = _mm(rel_ref[b], w2_ref[...])
        a2_b = a2_ref[b]
        fr2_row = jnp.sum(frT_b * frT_b, axis=0, keepdims=True)
        fa1sq_row = jnp.sum(fa1T_b * fa1T_b, axis=0, keepdims=True)
        fa2sq_row = jnp.sum(fa2T_b * fa2T_b, axis=0, keepdims=True)
        cap2_row = jnp.maximum(d2_row(hop2_b, frT_b, fr2_row),
                               d2_row(a2_b, fa2T_b, fa2sq_row))
        cap2_row = jnp.where(lane_f < nb_b, cap2_row, inf)

        onehot_b = jnp.concatenate([ohs[j][b:b + 1] for j in range(_BEAM)],
                                   axis=0)
        zembT_b = _mmt(entT_b, onehot_b)
        z2_row = jnp.sum(zembT_b * zembT_b, axis=0, keepdims=True)
        z2_col = _tm(z2_row, jnp.ones((1, 1), jnp.float32))
        xf2 = _tm(zembT_b, fa1T_b)
        d2z = jnp.maximum((z2_col + fa1sq_row) - 2.0 * xf2, 0.0)
        m2_rows.append(jnp.maximum(d2z, cap2_row))
        z_parts.extend(z_cols[j][b:b + 1] for j in range(_BEAM))

    m2_all = jnp.concatenate(m2_rows, axis=0)
    h2 = jnp.min(m2_all, axis=1, keepdims=True)
    z80 = jnp.concatenate(z_parts, axis=0)
    sc = jnp.minimum(jnp.exp(-0.5 * h2), z80)
    res_parts = [jnp.max(sc[b * _BEAM:(b + 1) * _BEAM]).reshape(1, 1)
                 for b in range(B)]
    res = jnp.concatenate(res_parts, axis=0)

    out_ref[...] = jnp.maximum(score0, res).reshape(B, 1, 1)


@jax.jit
def _run(nb_facts, rel, arg1, arg2, fact_rel, fact_arg1, fact_arg2, ent,
         W1, W2):
    B, E = rel.shape
    F = fact_rel.shape[1]
    N = ent.shape[1]
    full = lambda shape: pl.BlockSpec(shape, lambda i: (0,) * len(shape))
    out = pl.pallas_call(
        _prove_kernel,
        grid=(1,),
        in_specs=[
            pl.BlockSpec(memory_space=pltpu.SMEM),
            full((B, 1, E)),
            full((B, 1, E)),
            full((B, 1, E)),
            full((B, E, F)),
            full((B, E, F)),
            full((B, E, F)),
            full((B, E, N)),
            full((E, E)),
            full((E, E)),
        ],
        out_specs=full((B, 1, 1)),
        out_shape=jax.ShapeDtypeStruct((B, 1, 1), jnp.float32),
        scratch_shapes=[
            pltpu.VMEM((B, 1, F), jnp.float32),
            pltpu.VMEM((B, 1, N), jnp.float32),
        ],
        compiler_params=pltpu.CompilerParams(
            dimension_semantics=("arbitrary",),
            vmem_limit_bytes=52 * 1024 * 1024),
    )(nb_facts, rel[:, None, :], arg1[:, None, :], arg2[:, None, :],
      fact_rel.transpose(0, 2, 1), fact_arg1.transpose(0, 2, 1),
      fact_arg2.transpose(0, 2, 1), ent.transpose(0, 2, 1), W1, W2)
    return out[:, 0, 0]


def kernel(rel, arg1, arg2, fact_rel, fact_arg1, fact_arg2, nb_facts,
           entity_embeddings, nb_entities, W1, W2):
    return _run(nb_facts, rel, arg1, arg2, fact_rel, fact_arg1, fact_arg2,
                entity_embeddings, W1, W2)


# speedup vs baseline: 1.1094x; 1.1094x over previous
import jax
import jax.numpy as jnp
from jax.experimental import pallas as pl
from jax.experimental.pallas import tpu as pltpu

_BEAM = 10
_FC = 256


def _mm(a, b):
    return jax.lax.dot_general(a, b, (((1,), (0,)), ((), ())),
                               preferred_element_type=jnp.float32)


def _tm(a, b):
    return jax.lax.dot_general(a, b, (((0,), (0,)), ((), ())),
                               preferred_element_type=jnp.float32)


def _mmt(a, b):
    return jax.lax.dot_general(a, b, (((1,), (1,)), ((), ())),
                               preferred_element_type=jnp.float32)


def _prove_kernel(nb_s_ref, rel_ref, a1_ref, a2_ref, frT_ref,
                  fa1T_ref, fa2T_ref, entT_ref, w1_ref, w2_ref, out_ref,
                  s0_scr, dmin_scr):
    B, E, F = frT_ref.shape
    N = entT_ref.shape[2]
    inf = jnp.float32(jnp.inf)

    ones_row = jnp.ones((1, E), jnp.float32)
    ones_frow = jnp.ones((1, F), jnp.float32)
    ones_nrow = jnp.ones((1, N), jnp.float32)
    lane_f = jax.lax.broadcasted_iota(jnp.int32, (1, F), 1)

    def sq(x):
        return jnp.sum(x * x)

    def d2_row(x, fT, fsq_row):
        return jnp.maximum(fsq_row - 2.0 * _mm(x, fT) + sq(x), 0.0)

    def phase1_one(b):
        nb_b = nb_s_ref[b]
        rel_b = rel_ref[b]
        a1_b = a1_ref[b]
        a2_b = a2_ref[b]
        hop1_b = _mm(rel_b, w1_ref[...])
        frT_b = frT_ref[b]
        fa1T_b = fa1T_ref[b]
        fa2T_b = fa2T_ref[b]
        entT_b = entT_ref[b]
        valid_row = lane_f < nb_b

        fr2_row = _mm(ones_row, frT_b * frT_b)
        fa1sq_row = _mm(ones_row, fa1T_b * fa1T_b)
        fa2sq_row = _mm(ones_row, fa2T_b * fa2T_b)

        xf2r = _mm(jnp.concatenate([rel_b, hop1_b], axis=0), frT_b)
        d2_rel = jnp.maximum(fr2_row - 2.0 * xf2r[0:1] + sq(rel_b), 0.0)
        d2_hop1 = jnp.maximum(fr2_row - 2.0 * xf2r[1:2] + sq(hop1_b), 0.0)
        d2_a1 = d2_row(a1_b, fa1T_b, fa1sq_row)
        d2_a2 = d2_row(a2_b, fa2T_b, fa2sq_row)

        s0_row = jnp.maximum(jnp.maximum(d2_rel, d2_a1), d2_a2)
        s0_row = jnp.where(valid_row, s0_row, inf)

        cap1_row = jnp.maximum(d2_hop1, d2_a1)
        cap1_row = jnp.where(valid_row, cap1_row, inf)
        cap1_col = _tm(cap1_row, jnp.ones((1, 1), jnp.float32))

        e2_row = _mm(ones_row, entT_b * entT_b)
        rhs_aug = jnp.concatenate([entT_b, ones_nrow, e2_row], axis=0)
        lhs_aug = jnp.concatenate([-2.0 * fa2T_b, fa2sq_row, ones_frow],
                                  axis=0)

        def chunk_min(t):
            pre = _tm(lhs_aug[:, t * _FC:(t + 1) * _FC], rhs_aug)
            m = jnp.maximum(pre, cap1_col[t * _FC:(t + 1) * _FC])
            return jnp.min(m, axis=0, keepdims=True)

        s0_scr[b] = s0_row
        dmin_scr[b] = chunk_min(0)
        for t in range(1, F // _FC):
            @pl.when(nb_b > t * _FC)
            def _(t=t):
                dmin_scr[b] = jnp.minimum(dmin_scr[b], chunk_min(t))

    def phase1_body(b, carry):
        phase1_one(b)
        return carry

    jax.lax.fori_loop(0, B, phase1_body, 0)
    s0_all = jnp.concatenate([s0_scr[b] for b in range(B)], axis=0)
    dmin_all = jnp.concatenate([dmin_scr[b] for b in range(B)], axis=0)
    score0 = jnp.exp(-0.5 * jnp.min(s0_all, axis=1, keepdims=True))

    vals = jnp.exp(-0.5 * dmin_all)

    lane_n = jax.lax.broadcasted_iota(jnp.int32, (B, N), 1)
    ohs = []
    z_cols = []
    v = vals
    for _ in range(_BEAM):
        mv = jnp.max(v, axis=1, keepdims=True)
        idx = jnp.min(jnp.where(v == mv, lane_n, N), axis=1,
                      keepdims=True)
        oh = lane_n == idx
        v = jnp.where(oh, -inf, v)
        z_cols.append(mv)
        ohs.append(oh.astype(jnp.float32))

    ones_krow = jnp.ones((1, _BEAM), jnp.float32)
    m2_rows = []
    z_parts = []
    for b in range(B):
        nb_b = nb_s_ref[b]
        frT_b = frT_ref[b]
        fa1T_b = fa1T_ref[b]
        fa2T_b = fa2T_ref[b]
        entT_b = entT_ref[b]
        hop2_b = _mm(rel_ref[b], w2_ref[...])
        a2_b = a2_ref[b]
        fr2_row = _mm(ones_row, frT_b * frT_b)
        fa1sq_row = _mm(ones_row, fa1T_b * fa1T_b)
        fa2sq_row = _mm(ones_row, fa2T_b * fa2T_b)
        cap2_row = jnp.maximum(d2_row(hop2_b, frT_b, fr2_row),
                               d2_row(a2_b, fa2T_b, fa2sq_row))
        cap2_row = jnp.where(lane_f < nb_b, cap2_row, inf)

        onehot_b = jnp.concatenate([ohs[j][b:b + 1] for j in range(_BEAM)],
                                   axis=0)
        zembT_b = _mmt(entT_b, onehot_b)
        z2_row = _mm(ones_row, zembT_b * zembT_b)
        lhs2 = jnp.concatenate([-2.0 * zembT_b, z2_row, ones_krow],
                               axis=0)
        rhs2 = jnp.concatenate([fa1T_b, ones_frow, fa1sq_row],
                               axis=0)
        pre2 = _tm(lhs2, rhs2)
        m2_rows.append(jnp.maximum(pre2, cap2_row))
        z_parts.extend(z_cols[j][b:b + 1] for j in range(_BEAM))

    m2_all = jnp.concatenate(m2_rows, axis=0)
    h2 = jnp.min(m2_all, axis=1, keepdims=True)
    z80 = jnp.concatenate(z_parts, axis=0)
    sc = jnp.minimum(jnp.exp(-0.5 * h2), z80)
    res_parts = [jnp.max(sc[b * _BEAM:(b + 1) * _BEAM]).reshape(1, 1)
                 for b in range(B)]
    res = jnp.concatenate(res_parts, axis=0)

    out_ref[...] = jnp.maximum(score0, res).reshape(B, 1, 1)


@jax.jit
def _run(nb_facts, rel, arg1, arg2, fact_rel, fact_arg1, fact_arg2, ent,
         W1, W2):
    B, E = rel.shape
    F = fact_rel.shape[1]
    N = ent.shape[1]
    full = lambda shape: pl.BlockSpec(shape, lambda i: (0,) * len(shape))
    out = pl.pallas_call(
        _prove_kernel,
        grid=(1,),
        in_specs=[
            pl.BlockSpec(memory_space=pltpu.SMEM),
            full((B, 1, E)),
            full((B, 1, E)),
            full((B, 1, E)),
            full((B, E, F)),
            full((B, E, F)),
            full((B, E, F)),
            full((B, E, N)),
            full((E, E)),
            full((E, E)),
        ],
        out_specs=full((B, 1, 1)),
        out_shape=jax.ShapeDtypeStruct((B, 1, 1), jnp.float32),
        scratch_shapes=[
            pltpu.VMEM((B, 1, F), jnp.float32),
            pltpu.VMEM((B, 1, N), jnp.float32),
        ],
        compiler_params=pltpu.CompilerParams(
            dimension_semantics=("arbitrary",),
            vmem_limit_bytes=52 * 1024 * 1024),
    )(nb_facts, rel[:, None, :], arg1[:, None, :], arg2[:, None, :],
      fact_rel.transpose(0, 2, 1), fact_arg1.transpose(0, 2, 1),
      fact_arg2.transpose(0, 2, 1), ent.transpose(0, 2, 1), W1, W2)
    return out[:, 0, 0]


def kernel(rel, arg1, arg2, fact_rel, fact_arg1, fact_arg2, nb_facts,
           entity_embeddings, nb_entities, W1, W2):
    return _run(nb_facts, rel, arg1, arg2, fact_rel, fact_arg1, fact_arg2,
                entity_embeddings, W1, W2)
